# baseline (speedup 1.0000x reference)
import functools

import jax
import jax.numpy as jnp
from jax import lax
from jax.experimental import pallas as pl
from jax.experimental.pallas import tpu as pltpu
from jax.experimental.pallas import tpu_sc as plsc

B = 8
N = 8192
G = 512
M = 32
D = 128


def _fps_body(x_ref, y_ref, z_ref, cx_ref, cy_ref, cz_ref):
    x = x_ref[...]
    y = y_ref[...]
    z = z_ref[...]
    lane = lax.broadcasted_iota(jnp.int32, (B, N), 1)
    glane = lax.broadcasted_iota(jnp.int32, (B, G), 1)

    def body(i, carry):
        dist, idx, cxs, cys, czs = carry
        sel = lane == idx
        cx = jnp.sum(jnp.where(sel, x, 0.0), axis=1, keepdims=True)
        cy = jnp.sum(jnp.where(sel, y, 0.0), axis=1, keepdims=True)
        cz = jnp.sum(jnp.where(sel, z, 0.0), axis=1, keepdims=True)
        rec = glane == i
        cxs = jnp.where(rec, cx, cxs)
        cys = jnp.where(rec, cy, cys)
        czs = jnp.where(rec, cz, czs)
        dx = x - cx
        dy = y - cy
        dz = z - cz
        d = dx * dx + dy * dy + dz * dz
        dist = jnp.minimum(dist, d)
        m = jnp.max(dist, axis=1, keepdims=True)
        nidx = jnp.min(jnp.where(dist == m, lane, N), axis=1, keepdims=True)
        return dist, nidx, cxs, cys, czs

    init = (
        jnp.full((B, N), jnp.inf, jnp.float32),
        jnp.zeros((B, 1), jnp.int32),
        jnp.zeros((B, G), jnp.float32),
        jnp.zeros((B, G), jnp.float32),
        jnp.zeros((B, G), jnp.float32),
    )
    _, _, cxs, cys, czs = lax.fori_loop(0, G, body, init)
    cx_ref[...] = cxs
    cy_ref[...] = cys
    cz_ref[...] = czs


def _fps(x, y, z):
    return pl.pallas_call(
        _fps_body,
        out_shape=[jax.ShapeDtypeStruct((B, G), jnp.float32)] * 3,
    )(x, y, z)


_CB = 128


def _knn_body(x_ref, y_ref, z_ref, cx_ref, cy_ref, cz_ref, idx_ref,
              nbx_ref, nby_ref, nbz_ref):
    b = pl.program_id(0)
    x = x_ref[0]
    y = y_ref[0]
    z = z_ref[0]
    bcol = lax.broadcasted_iota(jnp.int32, (_CB, B), 1) == b
    qx = jnp.sum(jnp.where(bcol, cx_ref[...], 0.0), axis=1, keepdims=True)
    qy = jnp.sum(jnp.where(bcol, cy_ref[...], 0.0), axis=1, keepdims=True)
    qz = jnp.sum(jnp.where(bcol, cz_ref[...], 0.0), axis=1, keepdims=True)
    rsq = x * x + y * y + z * z
    qsq = qx * qx + qy * qy + qz * qz
    qb = jnp.concatenate([qx, qy, qz], axis=1).astype(jnp.bfloat16)
    rb = jnp.concatenate([x, y, z], axis=0).astype(jnp.bfloat16)
    dot = jax.lax.dot_general(qb, rb, (((1,), (0,)), ((), ())),
                              preferred_element_type=jnp.float32)
    d = (qsq + rsq) - 2.0 * dot
    lane = lax.broadcasted_iota(jnp.int32, (_CB, N), 1)
    klane = lax.broadcasted_iota(jnp.int32, (_CB, M), 1)
    res = jnp.zeros((_CB, M), jnp.int32)
    nbx = jnp.zeros((_CB, M), jnp.float32)
    nby = jnp.zeros((_CB, M), jnp.float32)
    nbz = jnp.zeros((_CB, M), jnp.float32)
    for k in range(M):
        m = jnp.min(d, axis=1, keepdims=True)
        nidx = jnp.min(jnp.where(d == m, lane, N), axis=1, keepdims=True)
        oh = lane == nidx
        xj = jnp.sum(jnp.where(oh, x, 0.0), axis=1, keepdims=True)
        yj = jnp.sum(jnp.where(oh, y, 0.0), axis=1, keepdims=True)
        zj = jnp.sum(jnp.where(oh, z, 0.0), axis=1, keepdims=True)
        sel = klane == k
        res = jnp.where(sel, nidx, res)
        nbx = jnp.where(sel, xj - qx, nbx)
        nby = jnp.where(sel, yj - qy, nby)
        nbz = jnp.where(sel, zj - qz, nbz)
        d = jnp.where(oh, jnp.inf, d)
    idx_ref[...] = (res + b * N)[None]
    nbx_ref[...] = nbx[None]
    nby_ref[...] = nby[None]
    nbz_ref[...] = nbz[None]


def _knn(x, y, z, cxt, cyt, czt):
    grid = (B, G // _CB)
    xyz_spec = pl.BlockSpec((1, 1, N), lambda b, c: (b, 0, 0))
    ct_spec = pl.BlockSpec((_CB, B), lambda b, c: (c, 0))
    blk = pl.BlockSpec((1, _CB, M), lambda b, c: (b, c, 0))
    return pl.pallas_call(
        _knn_body,
        grid=grid,
        in_specs=[xyz_spec, xyz_spec, xyz_spec, ct_spec, ct_spec, ct_spec],
        out_specs=[blk, blk, blk, blk],
        out_shape=[
            jax.ShapeDtypeStruct((B, G, M), jnp.int32),
            jax.ShapeDtypeStruct((B, G, M), jnp.float32),
            jax.ShapeDtypeStruct((B, G, M), jnp.float32),
            jax.ShapeDtypeStruct((B, G, M), jnp.float32),
        ],
    )(x.reshape(B, 1, N), y.reshape(B, 1, N), z.reshape(B, 1, N), cxt, cyt, czt)


_ROWS = B * G * M
_NW = 32
_PER_W = _ROWS // _NW
_CHUNK = 128
_NC = _PER_W // _CHUNK


def _sc_gather(pts, flat_idx2d):
    mesh = plsc.VectorSubcoreMesh(core_axis_name="c", subcore_axis_name="s")

    @functools.partial(
        pl.kernel,
        mesh=mesh,
        out_type=jax.ShapeDtypeStruct((_ROWS, D), jnp.float32),
        scratch_types=[
            pltpu.VMEM((_NC, _CHUNK), jnp.int32),
            pltpu.VMEM((_CHUNK, D), jnp.float32),
            pltpu.VMEM((_CHUNK, D), jnp.float32),
            pltpu.SemaphoreType.DMA,
            pltpu.SemaphoreType.DMA,
        ],
    )
    def k(pts_hbm, idx_hbm, opts_hbm, idx_v, buf0, buf1, sem0, sem1):
        wid = lax.axis_index("s") * 2 + lax.axis_index("c")
        pltpu.sync_copy(idx_hbm.at[pl.ds(wid * _NC, _NC)], idx_v)
        bufs = (buf0, buf1)
        sems = (sem0, sem1)
        pltpu.async_copy(pts_hbm.at[idx_v.at[0]], buf0, sem0)

        def body(c, carry):
            for p in range(2):
                cc = c * 2 + p
                nxt = pltpu.async_copy(
                    pts_hbm.at[idx_v.at[cc + 1]], bufs[1 - p], sems[1 - p]
                )
                pltpu.make_async_copy(pts_hbm.at[idx_v.at[cc]], bufs[p], sems[p]).wait()
                base = wid * _PER_W + cc * _CHUNK
                pltpu.sync_copy(bufs[p], opts_hbm.at[pl.ds(base, _CHUNK)])
            return carry

        lax.fori_loop(0, (_NC - 2) // 2, body, 0)
        for cc in (_NC - 2, _NC - 1):
            p = cc % 2
            if cc + 1 < _NC:
                pltpu.async_copy(pts_hbm.at[idx_v.at[cc + 1]], bufs[1 - p], sems[1 - p])
            pltpu.make_async_copy(pts_hbm.at[idx_v.at[cc]], bufs[p], sems[p]).wait()
            base = wid * _PER_W + cc * _CHUNK
            pltpu.sync_copy(bufs[p], opts_hbm.at[pl.ds(base, _CHUNK)])

    return k(pts, flat_idx2d)


_AB = 2048


def _asm_body(bx_ref, by_ref, bz_ref, pg_ref, nb_ref, np_ref):
    nb = jnp.concatenate([bx_ref[...], by_ref[...], bz_ref[...]], axis=1)
    nb_ref[...] = nb
    np_ref[:, :3] = nb
    np_ref[:, 3:] = pg_ref[...]


def _assemble(nbx, nby, nbz, g_pts):
    grid = (_ROWS // _AB,)
    col = pl.BlockSpec((_AB, 1), lambda i: (i, 0))
    return pl.pallas_call(
        _asm_body,
        grid=grid,
        in_specs=[col, col, col, pl.BlockSpec((_AB, D), lambda i: (i, 0))],
        out_specs=[
            pl.BlockSpec((_AB, 3), lambda i: (i, 0)),
            pl.BlockSpec((_AB, 3 + D), lambda i: (i, 0)),
        ],
        out_shape=[
            jax.ShapeDtypeStruct((_ROWS, 3), jnp.float32),
            jax.ShapeDtypeStruct((_ROWS, 3 + D), jnp.float32),
        ],
    )(nbx, nby, nbz, g_pts)


def kernel(xyz, points):
    x = xyz[:, :, 0]
    y = xyz[:, :, 1]
    z = xyz[:, :, 2]
    cx, cy, cz = _fps(x, y, z)
    centers = jnp.stack([cx, cy, cz], axis=-1)
    idx, nbx, nby, nbz = _knn(x, y, z, cx.T, cy.T, cz.T)
    flat_idx2d = idx.reshape(_ROWS // _CHUNK, _CHUNK)
    g_pts = _sc_gather(points.reshape(B * N, D), flat_idx2d)
    nb_flat, np_flat = _assemble(
        nbx.reshape(_ROWS, 1), nby.reshape(_ROWS, 1), nbz.reshape(_ROWS, 1), g_pts
    )
    neighborhood = nb_flat.reshape(B, G, M, 3)
    new_points = np_flat.reshape(B, G, M, 3 + D)
    return neighborhood, new_points, centers


# speedup vs baseline: 1.0584x; 1.0584x over previous
import functools

import jax
import jax.numpy as jnp
from jax import lax
from jax.experimental import pallas as pl
from jax.experimental.pallas import tpu as pltpu
from jax.experimental.pallas import tpu_sc as plsc

B = 8
N = 8192
G = 512
M = 32
D = 128


def _fps_body(x_ref, y_ref, z_ref, cx_ref, cy_ref, cz_ref):
    x = x_ref[...]
    y = y_ref[...]
    z = z_ref[...]
    lane = lax.broadcasted_iota(jnp.int32, (B, N), 1)
    glane = lax.broadcasted_iota(jnp.int32, (B, G), 1)

    def body(i, carry):
        dist, idx, cxs, cys, czs = carry
        sel = lane == idx
        cx = jnp.sum(jnp.where(sel, x, 0.0), axis=1, keepdims=True)
        cy = jnp.sum(jnp.where(sel, y, 0.0), axis=1, keepdims=True)
        cz = jnp.sum(jnp.where(sel, z, 0.0), axis=1, keepdims=True)
        rec = glane == i
        cxs = jnp.where(rec, cx, cxs)
        cys = jnp.where(rec, cy, cys)
        czs = jnp.where(rec, cz, czs)
        dx = x - cx
        dy = y - cy
        dz = z - cz
        d = dx * dx + dy * dy + dz * dz
        dist = jnp.minimum(dist, d)
        m = jnp.max(dist, axis=1, keepdims=True)
        nidx = jnp.min(jnp.where(dist == m, lane, N), axis=1, keepdims=True)
        return dist, nidx, cxs, cys, czs

    init = (
        jnp.full((B, N), jnp.inf, jnp.float32),
        jnp.zeros((B, 1), jnp.int32),
        jnp.zeros((B, G), jnp.float32),
        jnp.zeros((B, G), jnp.float32),
        jnp.zeros((B, G), jnp.float32),
    )
    _, _, cxs, cys, czs = lax.fori_loop(0, G, body, init)
    cx_ref[...] = cxs
    cy_ref[...] = cys
    cz_ref[...] = czs


def _fps(x, y, z):
    return pl.pallas_call(
        _fps_body,
        out_shape=[jax.ShapeDtypeStruct((B, G), jnp.float32)] * 3,
    )(x, y, z)


_CB = 128


def _knn_body(x_ref, y_ref, z_ref, cx_ref, cy_ref, cz_ref, idx_ref,
              nbx_ref, nby_ref, nbz_ref):
    b = pl.program_id(0)
    x = x_ref[0]
    y = y_ref[0]
    z = z_ref[0]
    bcol = lax.broadcasted_iota(jnp.int32, (_CB, B), 1) == b
    qx = jnp.sum(jnp.where(bcol, cx_ref[...], 0.0), axis=1, keepdims=True)
    qy = jnp.sum(jnp.where(bcol, cy_ref[...], 0.0), axis=1, keepdims=True)
    qz = jnp.sum(jnp.where(bcol, cz_ref[...], 0.0), axis=1, keepdims=True)
    rsq = x * x + y * y + z * z
    qsq = qx * qx + qy * qy + qz * qz
    qb = jnp.concatenate([qx, qy, qz], axis=1).astype(jnp.bfloat16)
    rb = jnp.concatenate([x, y, z], axis=0).astype(jnp.bfloat16)
    dot = jax.lax.dot_general(qb, rb, (((1,), (0,)), ((), ())),
                              preferred_element_type=jnp.float32)
    d = (qsq + rsq) - 2.0 * dot
    lane = lax.broadcasted_iota(jnp.int32, (_CB, N), 1)
    klane = lax.broadcasted_iota(jnp.int32, (_CB, M), 1)
    res = jnp.zeros((_CB, M), jnp.int32)
    nbx = jnp.zeros((_CB, M), jnp.float32)
    nby = jnp.zeros((_CB, M), jnp.float32)
    nbz = jnp.zeros((_CB, M), jnp.float32)
    for k in range(M):
        m = jnp.min(d, axis=1, keepdims=True)
        nidx = jnp.min(jnp.where(d == m, lane, N), axis=1, keepdims=True)
        oh = lane == nidx
        xj = jnp.sum(jnp.where(oh, x, 0.0), axis=1, keepdims=True)
        yj = jnp.sum(jnp.where(oh, y, 0.0), axis=1, keepdims=True)
        zj = jnp.sum(jnp.where(oh, z, 0.0), axis=1, keepdims=True)
        sel = klane == k
        res = jnp.where(sel, nidx, res)
        nbx = jnp.where(sel, xj - qx, nbx)
        nby = jnp.where(sel, yj - qy, nby)
        nbz = jnp.where(sel, zj - qz, nbz)
        d = jnp.where(oh, jnp.inf, d)
    idx_ref[...] = (res + b * N)[None]
    nbx_ref[...] = nbx[None]
    nby_ref[...] = nby[None]
    nbz_ref[...] = nbz[None]


def _knn(x, y, z, cxt, cyt, czt):
    grid = (B, G // _CB)
    xyz_spec = pl.BlockSpec((1, 1, N), lambda b, c: (b, 0, 0))
    ct_spec = pl.BlockSpec((_CB, B), lambda b, c: (c, 0))
    blk = pl.BlockSpec((1, _CB, M), lambda b, c: (b, c, 0))
    return pl.pallas_call(
        _knn_body,
        grid=grid,
        in_specs=[xyz_spec, xyz_spec, xyz_spec, ct_spec, ct_spec, ct_spec],
        out_specs=[blk, blk, blk, blk],
        out_shape=[
            jax.ShapeDtypeStruct((B, G, M), jnp.int32),
            jax.ShapeDtypeStruct((B, G, M), jnp.float32),
            jax.ShapeDtypeStruct((B, G, M), jnp.float32),
            jax.ShapeDtypeStruct((B, G, M), jnp.float32),
        ],
    )(x.reshape(B, 1, N), y.reshape(B, 1, N), z.reshape(B, 1, N), cxt, cyt, czt)


_ROWS = B * G * M
_NW = 32
_PER_W = _ROWS // _NW
_CHUNK = 128
_NC = _PER_W // _CHUNK


def _sc_gather(pts, flat_idx2d):
    mesh = plsc.VectorSubcoreMesh(core_axis_name="c", subcore_axis_name="s")

    @functools.partial(
        pl.kernel,
        mesh=mesh,
        out_type=jax.ShapeDtypeStruct((_ROWS, D), jnp.float32),
        scratch_types=[
            pltpu.VMEM((_NC, _CHUNK), jnp.int32),
            pltpu.VMEM((_CHUNK, D), jnp.float32),
            pltpu.VMEM((_CHUNK, D), jnp.float32),
            pltpu.SemaphoreType.DMA,
            pltpu.SemaphoreType.DMA,
        ],
    )
    def k(pts_hbm, idx_hbm, opts_hbm, idx_v, buf0, buf1, sem0, sem1):
        wid = lax.axis_index("s") * 2 + lax.axis_index("c")
        pltpu.sync_copy(idx_hbm.at[pl.ds(wid * _NC, _NC)], idx_v)
        bufs = (buf0, buf1)
        sems = (sem0, sem1)
        pltpu.async_copy(pts_hbm.at[idx_v.at[0]], buf0, sem0)

        def body(c, carry):
            for p in range(2):
                cc = c * 2 + p
                nxt = pltpu.async_copy(
                    pts_hbm.at[idx_v.at[cc + 1]], bufs[1 - p], sems[1 - p]
                )
                pltpu.make_async_copy(pts_hbm.at[idx_v.at[cc]], bufs[p], sems[p]).wait()
                base = wid * _PER_W + cc * _CHUNK
                pltpu.sync_copy(bufs[p], opts_hbm.at[pl.ds(base, _CHUNK)])
            return carry

        lax.fori_loop(0, (_NC - 2) // 2, body, 0)
        for cc in (_NC - 2, _NC - 1):
            p = cc % 2
            if cc + 1 < _NC:
                pltpu.async_copy(pts_hbm.at[idx_v.at[cc + 1]], bufs[1 - p], sems[1 - p])
            pltpu.make_async_copy(pts_hbm.at[idx_v.at[cc]], bufs[p], sems[p]).wait()
            base = wid * _PER_W + cc * _CHUNK
            pltpu.sync_copy(bufs[p], opts_hbm.at[pl.ds(base, _CHUNK)])

    return k(pts, flat_idx2d)


_AB = 2048


_AG = _AB // M


def _asm_body(bx_ref, by_ref, bz_ref, pg_ref, nb_ref, np_ref):
    bx = bx_ref[...]
    by = by_ref[...]
    bz = bz_ref[...]
    nb_ref[:, :, 0] = bx
    nb_ref[:, :, 1] = by
    nb_ref[:, :, 2] = bz
    np_ref[:, :, 0] = bx
    np_ref[:, :, 1] = by
    np_ref[:, :, 2] = bz
    np_ref[:, :, 3:] = pg_ref[...]


def _assemble(nbx, nby, nbz, g_pts):
    grid = (_ROWS // _AB,)
    col = pl.BlockSpec((_AG, M), lambda i: (i, 0))
    return pl.pallas_call(
        _asm_body,
        grid=grid,
        in_specs=[col, col, col,
                  pl.BlockSpec((_AG, M, D), lambda i: (i, 0, 0))],
        out_specs=[
            pl.BlockSpec((_AG, M, 3), lambda i: (i, 0, 0)),
            pl.BlockSpec((_AG, M, 3 + D), lambda i: (i, 0, 0)),
        ],
        out_shape=[
            jax.ShapeDtypeStruct((B * G, M, 3), jnp.float32),
            jax.ShapeDtypeStruct((B * G, M, 3 + D), jnp.float32),
        ],
    )(nbx, nby, nbz, g_pts.reshape(B * G, M, D))


def kernel(xyz, points):
    x = xyz[:, :, 0]
    y = xyz[:, :, 1]
    z = xyz[:, :, 2]
    cx, cy, cz = _fps(x, y, z)
    centers = jnp.stack([cx, cy, cz], axis=-1)
    idx, nbx, nby, nbz = _knn(x, y, z, cx.T, cy.T, cz.T)
    flat_idx2d = idx.reshape(_ROWS // _CHUNK, _CHUNK)
    g_pts = _sc_gather(points.reshape(B * N, D), flat_idx2d)
    nb_flat, np_flat = _assemble(
        nbx.reshape(B * G, M), nby.reshape(B * G, M), nbz.reshape(B * G, M), g_pts
    )
    neighborhood = nb_flat.reshape(B, G, M, 3)
    new_points = np_flat.reshape(B, G, M, 3 + D)
    return neighborhood, new_points, centers


# speedup vs baseline: 2.1096x; 1.9932x over previous
import functools

import jax
import jax.numpy as jnp
from jax import lax
from jax.experimental import pallas as pl
from jax.experimental.pallas import tpu as pltpu
from jax.experimental.pallas import tpu_sc as plsc

B = 8
N = 8192
G = 512
M = 32
D = 128


def _fps_body(x_ref, y_ref, z_ref, cx_ref, cy_ref, cz_ref):
    x = x_ref[...]
    y = y_ref[...]
    z = z_ref[...]
    lane = lax.broadcasted_iota(jnp.int32, (B, N), 1)
    glane = lax.broadcasted_iota(jnp.int32, (B, G), 1)

    def body(i, carry):
        dist, idx, cxs, cys, czs = carry
        sel = lane == idx
        cx = jnp.sum(jnp.where(sel, x, 0.0), axis=1, keepdims=True)
        cy = jnp.sum(jnp.where(sel, y, 0.0), axis=1, keepdims=True)
        cz = jnp.sum(jnp.where(sel, z, 0.0), axis=1, keepdims=True)
        rec = glane == i
        cxs = jnp.where(rec, cx, cxs)
        cys = jnp.where(rec, cy, cys)
        czs = jnp.where(rec, cz, czs)
        dx = x - cx
        dy = y - cy
        dz = z - cz
        d = dx * dx + dy * dy + dz * dz
        dist = jnp.minimum(dist, d)
        m = jnp.max(dist, axis=1, keepdims=True)
        nidx = jnp.min(jnp.where(dist == m, lane, N), axis=1, keepdims=True)
        return dist, nidx, cxs, cys, czs

    init = (
        jnp.full((B, N), jnp.inf, jnp.float32),
        jnp.zeros((B, 1), jnp.int32),
        jnp.zeros((B, G), jnp.float32),
        jnp.zeros((B, G), jnp.float32),
        jnp.zeros((B, G), jnp.float32),
    )
    _, _, cxs, cys, czs = lax.fori_loop(0, G, body, init)
    cx_ref[...] = cxs
    cy_ref[...] = cys
    cz_ref[...] = czs


def _fps(x, y, z):
    return pl.pallas_call(
        _fps_body,
        out_shape=[jax.ShapeDtypeStruct((B, G), jnp.float32)] * 3,
    )(x, y, z)


_CB = 128


def _knn_body(x_ref, y_ref, z_ref, cx_ref, cy_ref, cz_ref, idx_ref):
    b = pl.program_id(0)
    x = x_ref[0]
    y = y_ref[0]
    z = z_ref[0]
    bcol = lax.broadcasted_iota(jnp.int32, (_CB, B), 1) == b
    qx = jnp.sum(jnp.where(bcol, cx_ref[...], 0.0), axis=1, keepdims=True)
    qy = jnp.sum(jnp.where(bcol, cy_ref[...], 0.0), axis=1, keepdims=True)
    qz = jnp.sum(jnp.where(bcol, cz_ref[...], 0.0), axis=1, keepdims=True)
    rsq = x * x + y * y + z * z
    qsq = qx * qx + qy * qy + qz * qz
    qb = jnp.concatenate([qx, qy, qz], axis=1).astype(jnp.bfloat16)
    rb = jnp.concatenate([x, y, z], axis=0).astype(jnp.bfloat16)
    dot = jax.lax.dot_general(qb, rb, (((1,), (0,)), ((), ())),
                              preferred_element_type=jnp.float32)
    d = (qsq + rsq) - 2.0 * dot
    lane = lax.broadcasted_iota(jnp.int32, (_CB, N), 1)
    klane = lax.broadcasted_iota(jnp.int32, (_CB, M), 1)
    res = jnp.zeros((_CB, M), jnp.int32)
    for k in range(M):
        m = jnp.min(d, axis=1, keepdims=True)
        nidx = jnp.min(jnp.where(d == m, lane, N), axis=1, keepdims=True)
        res = jnp.where(klane == k, nidx, res)
        d = jnp.where(lane == nidx, jnp.inf, d)
    idx_ref[...] = (res + b * N)[None]


def _knn(x, y, z, cxt, cyt, czt):
    grid = (B, G // _CB)
    xyz_spec = pl.BlockSpec((1, 1, N), lambda b, c: (b, 0, 0))
    ct_spec = pl.BlockSpec((_CB, B), lambda b, c: (c, 0))
    return pl.pallas_call(
        _knn_body,
        grid=grid,
        in_specs=[xyz_spec, xyz_spec, xyz_spec, ct_spec, ct_spec, ct_spec],
        out_specs=pl.BlockSpec((1, _CB, M), lambda b, c: (b, c, 0)),
        out_shape=jax.ShapeDtypeStruct((B, G, M), jnp.int32),
    )(x.reshape(B, 1, N), y.reshape(B, 1, N), z.reshape(B, 1, N), cxt, cyt, czt)


_ROWS = B * G * M
_NW = 32
_PER_W = _ROWS // _NW
_CHUNK = 128
_NC = _PER_W // _CHUNK


_XW = 128
_XO = 128


def _sc_gather(xyz_pad, pts, flat_idx2d):
    mesh = plsc.VectorSubcoreMesh(core_axis_name="c", subcore_axis_name="s")

    @functools.partial(
        pl.kernel,
        mesh=mesh,
        out_type=[
            jax.ShapeDtypeStruct((_ROWS, _XO), jnp.float32),
            jax.ShapeDtypeStruct((_ROWS, D), jnp.float32),
        ],
        scratch_types=[
            pltpu.VMEM((_NC, _CHUNK), jnp.int32),
            pltpu.VMEM((_CHUNK, _XW), jnp.float32),
            pltpu.VMEM((_CHUNK, _XW), jnp.float32),
            pltpu.VMEM((_CHUNK, D), jnp.float32),
            pltpu.VMEM((_CHUNK, D), jnp.float32),
            pltpu.SemaphoreType.DMA,
            pltpu.SemaphoreType.DMA,
        ],
    )
    def k(xyz_hbm, pts_hbm, idx_hbm, oxyz_hbm, opts_hbm,
          idx_v, xb0, xb1, pb0, pb1, sem0, sem1):
        wid = lax.axis_index("s") * 2 + lax.axis_index("c")
        pltpu.sync_copy(idx_hbm.at[pl.ds(wid * _NC, _NC)], idx_v)
        xbufs = (xb0, xb1)
        pbufs = (pb0, pb1)
        sems = (sem0, sem1)

        def issue(cc, p):
            pltpu.async_copy(xyz_hbm.at[idx_v.at[cc]], xbufs[p], sems[p])
            pltpu.async_copy(pts_hbm.at[idx_v.at[cc]], pbufs[p], sems[p])

        def drain_and_store(cc, p):
            pltpu.make_async_copy(xyz_hbm.at[idx_v.at[cc]], xbufs[p], sems[p]).wait()
            pltpu.make_async_copy(pts_hbm.at[idx_v.at[cc]], pbufs[p], sems[p]).wait()
            base = wid * _PER_W + cc * _CHUNK
            pltpu.sync_copy(xbufs[p], oxyz_hbm.at[pl.ds(base, _CHUNK)])
            pltpu.sync_copy(pbufs[p], opts_hbm.at[pl.ds(base, _CHUNK)])

        issue(0, 0)

        def body(c, carry):
            for p in range(2):
                cc = c * 2 + p
                issue(cc + 1, 1 - p)
                drain_and_store(cc, p)
            return carry

        lax.fori_loop(0, (_NC - 2) // 2, body, 0)
        for cc in (_NC - 2, _NC - 1):
            p = cc % 2
            if cc + 1 < _NC:
                issue(cc + 1, 1 - p)
            drain_and_store(cc, p)

    return k(xyz_pad, pts, flat_idx2d)


_AB = 2048


_AG = _AB // M


def _asm_body(xg_ref, pg_ref, ct_ref, nb_ref, np_ref):
    nb = xg_ref[:, :, :3] - ct_ref[...][:, None, :]
    nb_ref[...] = nb
    np_ref[:, :, :3] = nb
    np_ref[:, :, 3:] = pg_ref[...]


def _assemble(g_xyz, g_pts, centers_flat):
    grid = (_ROWS // _AB,)
    return pl.pallas_call(
        _asm_body,
        grid=grid,
        in_specs=[
            pl.BlockSpec((_AG, M, _XO), lambda i: (i, 0, 0)),
            pl.BlockSpec((_AG, M, D), lambda i: (i, 0, 0)),
            pl.BlockSpec((_AG, 3), lambda i: (i, 0)),
        ],
        out_specs=[
            pl.BlockSpec((_AG, M, 3), lambda i: (i, 0, 0)),
            pl.BlockSpec((_AG, M, 3 + D), lambda i: (i, 0, 0)),
        ],
        out_shape=[
            jax.ShapeDtypeStruct((B * G, M, 3), jnp.float32),
            jax.ShapeDtypeStruct((B * G, M, 3 + D), jnp.float32),
        ],
    )(g_xyz.reshape(B * G, M, _XO), g_pts.reshape(B * G, M, D), centers_flat)


def kernel(xyz, points):
    x = xyz[:, :, 0]
    y = xyz[:, :, 1]
    z = xyz[:, :, 2]
    cx, cy, cz = _fps(x, y, z)
    centers = jnp.stack([cx, cy, cz], axis=-1)
    idx = _knn(x, y, z, cx.T, cy.T, cz.T)
    flat_idx2d = idx.reshape(_ROWS // _CHUNK, _CHUNK)
    xyz_pad = jnp.pad(xyz.reshape(B * N, 3), ((0, 0), (0, _XW - 3)))
    g_xyz, g_pts = _sc_gather(xyz_pad, points.reshape(B * N, D), flat_idx2d)
    nb_flat, np_flat = _assemble(g_xyz, g_pts, centers.reshape(B * G, 3))
    neighborhood = nb_flat.reshape(B, G, M, 3)
    new_points = np_flat.reshape(B, G, M, 3 + D)
    return neighborhood, new_points, centers
